# Initial kernel scaffold; baseline (speedup 1.0000x reference)
import jax
import jax.numpy as jnp
from jax.experimental import pallas as pl


def kernel(x, edge_index, W0, b0, W1, b1, W2, b2):
    raise NotImplementedError("write your pallas kernel here")


# speedup vs baseline: 8.4815x; 8.4815x over previous
import functools

import jax
import jax.numpy as jnp
from jax import lax
from jax.experimental import pallas as pl
from jax.experimental.pallas import tpu as pltpu
from jax.experimental.pallas import tpu_sc as plsc

N = 10000
D = 128
NC = 2
NS = 16
NW = NC * NS
CH = 128
NPAD = 10240
RPT = NPAD // NS
DG = 16

_mesh = plsc.VectorSubcoreMesh(core_axis_name="c", subcore_axis_name="s")


def _deg_body(dst_hbm, ones_hbm, zeros_hbm, out_hbm, didx, ones_v, zb, dacc):
    c = lax.axis_index("c")
    s = lax.axis_index("s")
    wid = s * NC + c
    cpw = dst_hbm.shape[0] // (NW * CH)
    pltpu.sync_copy(ones_hbm, ones_v)
    pltpu.sync_copy(zeros_hbm, zb)
    for k in range(RPT // CH):
        pltpu.sync_copy(zb, dacc.at[pl.ds(s * RPT + k * CH, CH), :])
    plsc.subcore_barrier()

    base = wid * cpw * CH

    def body(g, carry):
        pltpu.sync_copy(dst_hbm.at[pl.ds(base + g * CH, CH)], didx)
        pltpu.sync_copy(ones_v, dacc.at[didx], add=True)
        return carry

    lax.fori_loop(0, cpw, body, 0)
    plsc.subcore_barrier()
    for k in range(RPT // CH):
        r0 = s * RPT + k * CH
        pltpu.sync_copy(dacc.at[pl.ds(r0, CH), :], zb)
        pltpu.sync_copy(zb, out_hbm.at[c, pl.ds(r0, CH), :])


_deg_call = functools.partial(
    pl.kernel,
    out_type=jax.ShapeDtypeStruct((NC, NPAD, D), jnp.float32),
    mesh=_mesh,
    scratch_types=[
        pltpu.VMEM((CH,), jnp.int32),
        pltpu.VMEM((CH, D), jnp.float32),
        pltpu.VMEM((CH, D), jnp.float32),
        pltpu.VMEM_SHARED((NPAD, D), jnp.float32),
    ],
)(_deg_body)


def _agg_body(u_hbm, src_hbm, dst_hbm, zeros_hbm, out_hbm,
              sidx, didx, rows, zb, acc, sem):
    c = lax.axis_index("c")
    s = lax.axis_index("s")
    wid = s * NC + c
    cpw = src_hbm.shape[0] // (NW * CH)
    pltpu.sync_copy(zeros_hbm, zb)
    for k in range(RPT // CH):
        pltpu.sync_copy(zb, acc.at[pl.ds(s * RPT + k * CH, CH), :])
    plsc.subcore_barrier()

    base = wid * cpw * CH

    def body(g, carry):
        off = base + g * CH
        pltpu.sync_copy(src_hbm.at[pl.ds(off, CH)], sidx)
        pltpu.sync_copy(dst_hbm.at[pl.ds(off, CH)], didx)
        pltpu.async_copy(u_hbm.at[sidx], rows, sem).wait()
        pltpu.sync_copy(rows, acc.at[didx], add=True)
        return carry

    lax.fori_loop(0, cpw, body, 0)
    plsc.subcore_barrier()
    for k in range(RPT // CH):
        r0 = s * RPT + k * CH
        pltpu.sync_copy(acc.at[pl.ds(r0, CH), :], zb)
        pltpu.sync_copy(zb, out_hbm.at[c, pl.ds(r0, CH), :])


_agg_call = functools.partial(
    pl.kernel,
    out_type=jax.ShapeDtypeStruct((NC, NPAD, D), jnp.float32),
    mesh=_mesh,
    scratch_types=[
        pltpu.VMEM((CH,), jnp.int32),
        pltpu.VMEM((CH,), jnp.int32),
        pltpu.VMEM((CH, D), jnp.float32),
        pltpu.VMEM((CH, D), jnp.float32),
        pltpu.VMEM_SHARED((NPAD, D), jnp.float32),
        pltpu.SemaphoreType.DMA,
    ],
)(_agg_body)


def _dinv_from(degp_ref):
    deg = degp_ref[0][:, 0:1] + degp_ref[1][:, 0:1] + 1.0
    return lax.rsqrt(deg)


def _tc_first(xp_ref, w_ref, degp_ref, u_ref):
    dinv = _dinv_from(degp_ref)
    h = jnp.dot(xp_ref[...], w_ref[...], preferred_element_type=jnp.float32)
    u_ref[...] = h * dinv


def _pairnorm(h, mask):
    h = h * mask
    cm = jnp.sum(h, axis=0, keepdims=True) * (1.0 / N)
    xc = (h - cm) * mask
    r2 = jnp.sum(xc * xc) * (1.0 / N)
    return xc * lax.rsqrt(1e-6 + r2)


def _tc_mid(p_ref, u_ref, degp_ref, b_ref, w_ref, mask_ref, out_ref):
    dinv = _dinv_from(degp_ref)
    h = (p_ref[0] + p_ref[1] + u_ref[...]) * dinv + b_ref[...]
    y = jnp.maximum(_pairnorm(h, mask_ref[...]), 0.0)
    out_ref[...] = jnp.dot(
        y, w_ref[...], preferred_element_type=jnp.float32) * dinv


def _tc_last(p_ref, u_ref, degp_ref, b_ref, mask_ref, out_ref):
    dinv = _dinv_from(degp_ref)
    h = (p_ref[0] + p_ref[1] + u_ref[...]) * dinv + b_ref[...]
    out_ref[...] = _pairnorm(h, mask_ref[...])


_of32 = jax.ShapeDtypeStruct((NPAD, D), jnp.float32)
_tc_first_call = pl.pallas_call(_tc_first, out_shape=_of32)
_tc_mid_call = pl.pallas_call(_tc_mid, out_shape=_of32)
_tc_last_call = pl.pallas_call(_tc_last, out_shape=_of32)


@jax.jit
def _run(x, edge_index, W0, b0, W1, b1, W2, b2):
    E = edge_index.shape[1]
    cpw = -(-E // (NW * CH))
    epad = NW * cpw * CH
    fill = jnp.full((epad - E,), N, dtype=jnp.int32)
    srcp = jnp.concatenate([edge_index[0].astype(jnp.int32), fill])
    dstp = jnp.concatenate([edge_index[1].astype(jnp.int32), fill])

    xp = jnp.pad(x, ((0, NPAD - N), (0, 0)))
    W2p = jnp.pad(W2, ((0, 0), (0, D - W2.shape[1])))
    b0p = jnp.reshape(b0, (1, D))
    b1p = jnp.reshape(b1, (1, D))
    b2p = jnp.reshape(jnp.pad(b2, (0, D - b2.shape[0])), (1, D))
    mask = (jnp.arange(NPAD) < N).astype(jnp.float32)[:, None]
    zeros = jnp.zeros((CH, D), jnp.float32)
    ones = jnp.ones((CH, D), jnp.float32)

    degp = _deg_call(dstp, ones, zeros)

    u = _tc_first_call(xp, W0, degp)
    p = _agg_call(u, srcp, dstp, zeros)
    u = _tc_mid_call(p, u, degp, b0p, W1, mask)
    p = _agg_call(u, srcp, dstp, zeros)
    u = _tc_mid_call(p, u, degp, b1p, W2p, mask)
    p = _agg_call(u, srcp, dstp, zeros)
    out = _tc_last_call(p, u, degp, b2p, mask)
    return out[:N, :121]


def kernel(x, edge_index, W0, b0, W1, b1, W2, b2):
    return _run(x, edge_index, W0, b0, W1, b1, W2, b2)
